# baseline (speedup 1.0000x reference)
import functools

import jax
import jax.numpy as jnp
from jax import lax
from jax.experimental import pallas as pl
from jax.experimental.pallas import tpu as pltpu
from jax.experimental.pallas import tpu_sc as plsc

_NDIM = 6
_H = 10
_NUM_WORKERS = 32
_IDX_CHUNK = 128


def _tc_index_body(states_ref, idx_ref):
    x = states_ref[...]
    c = jnp.clip(jnp.floor(x * float(_H)), 0.0, float(_H - 1)).astype(jnp.int32)
    powers = (_H ** jnp.arange(_NDIM, dtype=jnp.int32)).reshape(1, _NDIM)
    idx_ref[...] = jnp.sum(c * powers, axis=1)


def _make_index_kernel(batch):
    return pl.pallas_call(
        _tc_index_body,
        out_shape=jax.ShapeDtypeStruct((batch,), jnp.int32),
    )


def _make_sc_gather(batch, n_states, out_dim):
    b_per_w = batch // _NUM_WORKERS
    n_idx_chunks = b_per_w // _IDX_CHUNK
    mesh = plsc.VectorSubcoreMesh(core_axis_name="c", subcore_axis_name="s")

    @functools.partial(
        pl.kernel,
        mesh=mesh,
        compiler_params=pltpu.CompilerParams(use_tc_tiling_on_sc=False),
        out_type=jax.ShapeDtypeStruct((batch, out_dim), jnp.float32),
        scratch_types=[
            pltpu.VMEM((n_idx_chunks, _IDX_CHUNK), jnp.int32),
            pltpu.VMEM((b_per_w, out_dim), jnp.float32),
            pltpu.SemaphoreType.DMA,
        ],
    )
    def sc_gather(idx_hbm, table_hbm, out_hbm, idx_v, rows_v, sem):
        wid = lax.axis_index("s") * 2 + lax.axis_index("c")
        base = wid * b_per_w
        for c in range(n_idx_chunks):
            pltpu.sync_copy(
                idx_hbm.at[pl.ds(base + c * _IDX_CHUNK, _IDX_CHUNK)],
                idx_v.at[c],
            )
        copies = []
        for c in range(n_idx_chunks):
            cp = pltpu.make_async_copy(
                table_hbm.at[idx_v.at[c]],
                rows_v.at[pl.ds(c * _IDX_CHUNK, _IDX_CHUNK)],
                sem,
            )
            cp.start()
            copies.append(cp)
        for cp in copies:
            cp.wait()
        pltpu.sync_copy(rows_v, out_hbm.at[pl.ds(base, b_per_w)])

    return sc_gather


def kernel(preprocessed_states, table):
    batch = preprocessed_states.shape[0]
    n_states, out_dim = table.shape
    idx = _make_index_kernel(batch)(preprocessed_states)
    return _make_sc_gather(batch, n_states, out_dim)(idx, table)
